# baseline (speedup 1.0000x reference)
import functools

import jax
import jax.numpy as jnp
from jax.experimental import pallas as pl
import jax.experimental.pallas.tpu as pltpu
from jax.experimental.pallas import tpu_sc as plsc

B, S, D = 1, 2048, 1024
H, DH = 16, 64
E, K = 8, 2
DFF, DSH = 512, 2048
EPS = 1e-6

BS = 256
NSB = S // BS
H2 = H // 2
DP = 2 * DH
BQ = 512
NQB = S // BQ

BLKA = 256
NBLK = (S * K) // BLKA + E
PAD = NBLK * BLKA
CH = 32


def _rms(x, g):
    return x * jax.lax.rsqrt(jnp.mean(x * x, axis=-1, keepdims=True) + EPS) * g


def _qkv_kernel(x_ref, g_ref, wq_ref, wk_ref, wv_ref, q_ref, k_ref, v_ref):
    h = _rms(x_ref[...], g_ref[...]).astype(jnp.bfloat16)
    q = jnp.dot(h, wq_ref[...].astype(jnp.bfloat16),
                preferred_element_type=jnp.float32).astype(jnp.bfloat16)
    k = jnp.dot(h, wk_ref[...].astype(jnp.bfloat16),
                preferred_element_type=jnp.float32).astype(jnp.bfloat16)
    v = jnp.dot(h, wv_ref[...].astype(jnp.bfloat16),
                preferred_element_type=jnp.float32).astype(jnp.bfloat16)
    q_ref[...] = q.reshape(BS, H2, DP).swapaxes(0, 1)
    k_ref[...] = k.reshape(BS, H2, DP).swapaxes(0, 1)
    v_ref[...] = v.reshape(BS, H2, DP).swapaxes(0, 1)


def _qkv(x, gamma, Wq, Wk, Wv):
    g2 = gamma.reshape(1, D)
    return pl.pallas_call(
        _qkv_kernel,
        grid=(NSB,),
        in_specs=[
            pl.BlockSpec((BS, D), lambda i: (i, 0)),
            pl.BlockSpec((1, D), lambda i: (0, 0)),
            pl.BlockSpec((D, H * DH), lambda i: (0, 0)),
            pl.BlockSpec((D, H * DH), lambda i: (0, 0)),
            pl.BlockSpec((D, H * DH), lambda i: (0, 0)),
        ],
        out_specs=[
            pl.BlockSpec((H2, BS, DP), lambda i: (0, i, 0)),
            pl.BlockSpec((H2, BS, DP), lambda i: (0, i, 0)),
            pl.BlockSpec((H2, BS, DP), lambda i: (0, i, 0)),
        ],
        out_shape=[jax.ShapeDtypeStruct((H2, S, DP), jnp.bfloat16)] * 3,
    )(x, g2, Wq, Wk, Wv)


def _attn_kernel(q_ref, k_ref, v_ref, o_ref, *, scale):
    i = pl.program_id(1)
    q2 = q_ref[0]
    lane = jax.lax.broadcasted_iota(jnp.int32, (BQ, DP), 1)
    is_a = lane < DH
    zero = jnp.zeros((), jnp.bfloat16)
    qa = jnp.where(is_a, q2, zero)
    qb = jnp.where(is_a, zero, q2)
    rows = jax.lax.broadcasted_iota(jnp.int32, (BQ, BQ), 0) + i * BQ

    def body(j, carry):
        ma, la, aa, mb, lb, ab = carry
        base = pl.multiple_of(j * BQ, BQ)
        k2 = k_ref[0, pl.ds(base, BQ), :]
        v2 = v_ref[0, pl.ds(base, BQ), :]
        cols = jax.lax.broadcasted_iota(jnp.int32, (BQ, BQ), 1) + j * BQ
        causal = cols <= rows

        def one(qh, m, l, acc):
            s = jax.lax.dot_general(qh, k2, (((1,), (1,)), ((), ())),
                                    preferred_element_type=jnp.float32) * scale
            s = jnp.where(causal, s, jnp.float32(-1e30))
            m_new = jnp.maximum(m, jnp.max(s, axis=-1, keepdims=True))
            alpha = jnp.exp(m - m_new)
            p = jnp.exp(s - m_new)
            l = l * alpha + jnp.sum(p, axis=-1, keepdims=True)
            acc = acc * alpha + jnp.dot(p.astype(jnp.bfloat16), v2,
                                        preferred_element_type=jnp.float32)
            return m_new, l, acc

        ma, la, aa = one(qa, ma, la, aa)
        mb, lb, ab = one(qb, mb, lb, ab)
        return ma, la, aa, mb, lb, ab

    m0 = jnp.full((BQ, 1), -1e30, jnp.float32)
    l0 = jnp.zeros((BQ, 1), jnp.float32)
    a0 = jnp.zeros((BQ, DP), jnp.float32)
    ma, la, aa, mb, lb, ab = jax.lax.fori_loop(
        0, i + 1, body, (m0, l0, a0, m0, l0, a0))
    oa = aa * (1.0 / la)
    ob = ab * (1.0 / lb)
    o_ref[0] = jnp.where(is_a, oa, ob).astype(jnp.bfloat16)


def _attention(q3, k3, v3):
    scale = 1.0 / float(DH) ** 0.5
    return pl.pallas_call(
        functools.partial(_attn_kernel, scale=scale),
        grid=(H2, NQB),
        in_specs=[
            pl.BlockSpec((1, BQ, DP), lambda h, i: (h, i, 0)),
            pl.BlockSpec((1, S, DP), lambda h, i: (h, 0, 0)),
            pl.BlockSpec((1, S, DP), lambda h, i: (h, 0, 0)),
        ],
        out_specs=pl.BlockSpec((1, BQ, DP), lambda h, i: (h, i, 0)),
        out_shape=jax.ShapeDtypeStruct((H2, S, DP), jnp.bfloat16),
    )(q3, k3, v3)


def _proj_router_kernel(a_ref, wo_ref, res_ref, g_ref, gw_ref,
                        h_ref, x2_ref, e12_ref, r12_ref, w12_ref, cnt_ref,
                        run_ref):
    i = pl.program_id(0)

    @pl.when(i == 0)
    def _():
        run_ref[...] = jnp.zeros_like(run_ref)

    attn = jnp.dot(a_ref[0], wo_ref[0].astype(jnp.bfloat16),
                   preferred_element_type=jnp.float32)
    for hh in range(1, H2):
        attn += jnp.dot(a_ref[hh], wo_ref[hh].astype(jnp.bfloat16),
                        preferred_element_type=jnp.float32)
    hstate = res_ref[...] + attn
    h_ref[...] = hstate
    x2 = _rms(hstate, g_ref[...])
    x2_ref[...] = x2.astype(jnp.bfloat16)
    logits = jnp.dot(x2, gw_ref[...], preferred_element_type=jnp.float32)
    m = jnp.max(logits, axis=-1, keepdims=True)
    p = jnp.exp(logits - m)
    p = p / jnp.sum(p, axis=-1, keepdims=True)
    idx = jax.lax.broadcasted_iota(jnp.int32, (BS, E), 1)
    m1 = jnp.max(p, axis=-1, keepdims=True)
    i1 = jnp.min(jnp.where(p == m1, idx, E), axis=-1, keepdims=True)
    p2 = jnp.where(idx == i1, -jnp.inf, p)
    m2 = jnp.max(p2, axis=-1, keepdims=True)
    i2 = jnp.min(jnp.where(p2 == m2, idx, E), axis=-1, keepdims=True)
    tot = m1 + m2
    oh = ((idx == i1) | (idx == i2)).astype(jnp.bfloat16)
    r_lo = jax.lax.broadcasted_iota(jnp.int32, (BS, BS), 0)
    c_lo = jax.lax.broadcasted_iota(jnp.int32, (BS, BS), 1)
    tril = (c_lo < r_lo).astype(jnp.bfloat16)
    prior = jnp.dot(tril, oh, preferred_element_type=jnp.float32)
    prior = prior + run_ref[...]
    rank1 = jnp.sum(jnp.where(idx == i1, prior, 0.0), axis=-1, keepdims=True)
    rank2 = jnp.sum(jnp.where(idx == i2, prior, 0.0), axis=-1, keepdims=True)
    run_ref[...] += jnp.sum(oh.astype(jnp.float32), axis=0, keepdims=True)
    cnt_ref[...] = run_ref[...]
    e12_ref[...] = jnp.concatenate([i1, i2], axis=1)
    r12_ref[...] = jnp.concatenate([rank1, rank2], axis=1).astype(jnp.int32)
    w12_ref[...] = jnp.concatenate([m1 / tot, m2 / tot], axis=1)


def _proj_router(attn, Wo, residual, gamma, gate_w):
    g2 = gamma.reshape(1, D)
    return pl.pallas_call(
        _proj_router_kernel,
        grid=(NSB,),
        in_specs=[
            pl.BlockSpec((H2, BS, DP), lambda i: (0, i, 0)),
            pl.BlockSpec((H2, DP, D), lambda i: (0, 0, 0)),
            pl.BlockSpec((BS, D), lambda i: (i, 0)),
            pl.BlockSpec((1, D), lambda i: (0, 0)),
            pl.BlockSpec((D, E), lambda i: (0, 0)),
        ],
        out_specs=[
            pl.BlockSpec((BS, D), lambda i: (i, 0)),
            pl.BlockSpec((BS, D), lambda i: (i, 0)),
            pl.BlockSpec((BS, 2), lambda i: (i, 0)),
            pl.BlockSpec((BS, 2), lambda i: (i, 0)),
            pl.BlockSpec((BS, 2), lambda i: (i, 0)),
            pl.BlockSpec((1, E), lambda i: (0, 0)),
        ],
        out_shape=[
            jax.ShapeDtypeStruct((S, D), jnp.float32),
            jax.ShapeDtypeStruct((S, D), jnp.bfloat16),
            jax.ShapeDtypeStruct((S, 2), jnp.int32),
            jax.ShapeDtypeStruct((S, 2), jnp.int32),
            jax.ShapeDtypeStruct((S, 2), jnp.float32),
            jax.ShapeDtypeStruct((1, E), jnp.float32),
        ],
        scratch_shapes=[pltpu.VMEM((1, E), jnp.float32)],
    )(attn, Wo.reshape(H2, DP, D), residual, g2, gate_w)


def _route_kernel(cnt_ref, e12_ref, r12_ref, dest_ref, be_ref):
    c = cnt_ref[...]
    padded = jnp.floor((c + (BLKA - 1)) / BLKA) * BLKA
    tri_r = jax.lax.broadcasted_iota(jnp.int32, (E, E), 0)
    tri_c = jax.lax.broadcasted_iota(jnp.int32, (E, E), 1)
    tril8 = (tri_r < tri_c).astype(jnp.float32)
    pad_start = jnp.dot(padded, tril8, preferred_element_type=jnp.float32)
    pad_end = pad_start + padded
    e12 = e12_ref[...]
    r12 = r12_ref[...]
    base = jnp.zeros((S, 2), jnp.float32)
    for ee in range(E):
        base = base + jnp.where(e12 == ee, pad_start[0, ee], 0.0)
    dest_ref[...] = (base + r12.astype(jnp.float32)).astype(jnp.int32)
    bvals = jax.lax.broadcasted_iota(jnp.int32, (1, NBLK), 1).astype(
        jnp.float32) * BLKA
    be = jnp.zeros((1, NBLK), jnp.int32)
    for ee in range(E):
        be = be + jnp.where(bvals >= pad_end[0, ee], 1, 0)
    be_ref[...] = jnp.minimum(be, E - 1)


def _route(counts, e12, r12):
    return pl.pallas_call(
        _route_kernel,
        out_shape=[
            jax.ShapeDtypeStruct((S, 2), jnp.int32),
            jax.ShapeDtypeStruct((1, NBLK), jnp.int32),
        ],
    )(counts, e12, r12)


def _sc_mesh():
    return plsc.VectorSubcoreMesh(core_axis_name="c", subcore_axis_name="s")


def _bf16_to_i32v(x):
    n, d = x.shape
    return jax.lax.bitcast_convert_type(
        x.reshape(n, d // 2, 2), jnp.int32).reshape(2 * n, d // 4)


def _i32v_to_bf16(x):
    n2, q = x.shape
    return jax.lax.bitcast_convert_type(
        x.reshape(n2 // 2, 2 * q), jnp.bfloat16).reshape(n2 // 2, 4 * q)


SCCH = 128
DQ = D // 4


def _dispatch(x2v, i0, i1):
    @pl.kernel(
        out_type=jax.ShapeDtypeStruct((2 * PAD, DQ), jnp.int32),
        mesh=_sc_mesh(),
    )
    def kern(x2_hbm, i0_hbm, i1_hbm, xs_hbm):
        def body(x_vmem, i0_vmem, i1_vmem):
            pltpu.sync_copy(x_vmem, xs_hbm.at[i0_vmem.at[0]])
            pltpu.sync_copy(x_vmem, xs_hbm.at[i1_vmem.at[0]])

        pltpu.emit_pipeline(
            body,
            grid=(2 * S // SCCH,),
            in_specs=[
                pl.BlockSpec((SCCH, DQ), lambda i: (i, 0)),
                pl.BlockSpec((1, SCCH), lambda i: (i, 0)),
                pl.BlockSpec((1, SCCH), lambda i: (i, 0)),
            ],
            out_specs=[],
            core_axis_name=("c", "s"),
            dimension_semantics=(pltpu.PARALLEL,),
        )(x2_hbm, i0_hbm, i1_hbm)

    return kern(x2v, i0, i1)


def _shared_kernel(x_ref, w1_ref, w3_ref, w2_ref, h_ref, o_ref):
    x = x_ref[...]
    g = jnp.dot(x, w1_ref[...].astype(jnp.bfloat16),
                preferred_element_type=jnp.float32)
    u = jnp.dot(x, w3_ref[...].astype(jnp.bfloat16),
                preferred_element_type=jnp.float32)
    a = (g * jax.lax.logistic(g) * u).astype(jnp.bfloat16)
    sh = jnp.dot(a, w2_ref[...].astype(jnp.bfloat16),
                 preferred_element_type=jnp.float32)
    o_ref[...] = h_ref[...] + sh


def _shared(x2, Ws1, Ws3, Ws2, hstate):
    return pl.pallas_call(
        _shared_kernel,
        grid=(NSB,),
        in_specs=[
            pl.BlockSpec((BS, D), lambda i: (i, 0)),
            pl.BlockSpec((D, DSH), lambda i: (0, 0)),
            pl.BlockSpec((D, DSH), lambda i: (0, 0)),
            pl.BlockSpec((DSH, D), lambda i: (0, 0)),
            pl.BlockSpec((BS, D), lambda i: (i, 0)),
        ],
        out_specs=pl.BlockSpec((BS, D), lambda i: (i, 0)),
        out_shape=jax.ShapeDtypeStruct((S, D), jnp.float32),
    )(x2, Ws1, Ws3, Ws2, hstate)


def _ffn_kernel(be_ref, xs_ref, wg_ref, wu_ref, wd_ref, ds_ref):
    x = xs_ref[...]
    g = jnp.dot(x, wg_ref[0].astype(jnp.bfloat16),
                preferred_element_type=jnp.float32)
    u = jnp.dot(x, wu_ref[0].astype(jnp.bfloat16),
                preferred_element_type=jnp.float32)
    a = (g * jax.lax.logistic(g) * u).astype(jnp.bfloat16)
    d = jnp.dot(a, wd_ref[0].astype(jnp.bfloat16),
                preferred_element_type=jnp.float32)
    ds_ref[...] = d.astype(jnp.bfloat16)


def _ffn(block_expert, xs, We_gate, We_up, We_down):
    grid_spec = pltpu.PrefetchScalarGridSpec(
        num_scalar_prefetch=1,
        grid=(NBLK,),
        in_specs=[
            pl.BlockSpec((BLKA, D), lambda b, be: (b, 0)),
            pl.BlockSpec((1, D, DFF), lambda b, be: (be[b], 0, 0)),
            pl.BlockSpec((1, D, DFF), lambda b, be: (be[b], 0, 0)),
            pl.BlockSpec((1, DFF, D), lambda b, be: (be[b], 0, 0)),
        ],
        out_specs=pl.BlockSpec((BLKA, D), lambda b, be: (b, 0)),
    )
    return pl.pallas_call(
        _ffn_kernel,
        grid_spec=grid_spec,
        out_shape=jax.ShapeDtypeStruct((PAD, D), jnp.bfloat16),
    )(block_expert, xs, We_gate, We_up, We_down)


def _gather_rows(dsv, idx):
    @pl.kernel(
        out_type=jax.ShapeDtypeStruct((2 * S, DQ), jnp.int32),
        mesh=_sc_mesh(),
    )
    def kern(ds_hbm, i_hbm, o_hbm):
        def body(i_vmem, o_vmem):
            pltpu.sync_copy(ds_hbm.at[i_vmem.at[0]], o_vmem)

        pltpu.emit_pipeline(
            body,
            grid=(2 * S // SCCH,),
            in_specs=[pl.BlockSpec((1, SCCH), lambda i: (i, 0))],
            out_specs=[pl.BlockSpec((SCCH, DQ), lambda i: (i, 0))],
            core_axis_name=("c", "s"),
            dimension_semantics=(pltpu.PARALLEL,),
        )(i_hbm, o_hbm)

    return kern(dsv, idx)


def _final_kernel(hs_ref, a_ref, b_ref, w0_ref, w1_ref, o_ref):
    o_ref[...] = hs_ref[...] + \
        w0_ref[...] * a_ref[...].astype(jnp.float32) + \
        w1_ref[...] * b_ref[...].astype(jnp.float32)


def _final(hs, A, Bm, w0, w1):
    return pl.pallas_call(
        _final_kernel,
        grid=(NSB,),
        in_specs=[
            pl.BlockSpec((BS, D), lambda i: (i, 0)),
            pl.BlockSpec((BS, D), lambda i: (i, 0)),
            pl.BlockSpec((BS, D), lambda i: (i, 0)),
            pl.BlockSpec((BS, 1), lambda i: (i, 0)),
            pl.BlockSpec((BS, 1), lambda i: (i, 0)),
        ],
        out_specs=pl.BlockSpec((BS, D), lambda i: (i, 0)),
        out_shape=jax.ShapeDtypeStruct((S, D), jnp.float32),
    )(hs, A, Bm, w0, w1)


def kernel(hidden_states, pre_ln_gamma, post_ln_gamma, Wq, Wk, Wv, Wo,
           gate_w, We_gate, We_up, We_down, Ws1, Ws3, Ws2):
    x = hidden_states.reshape(S, D)
    q3, k3, v3 = _qkv(x, pre_ln_gamma, Wq, Wk, Wv)
    attn = _attention(q3, k3, v3)
    hstate, x2, e12, r12, w12, counts = _proj_router(
        attn, Wo, x, post_ln_gamma, gate_w)
    dest, block_expert = _route(counts, e12, r12)
    d0 = dest[:, 0].reshape(1, S)
    d1 = dest[:, 1].reshape(1, S)
    w0 = w12[:, 0:1]
    w1 = w12[:, 1:2]
    two = jnp.int32(2)
    i0 = (dest[:, 0:1] * two + jnp.arange(2, dtype=jnp.int32)[None, :])
    i1 = (dest[:, 1:2] * two + jnp.arange(2, dtype=jnp.int32)[None, :])
    i0 = i0.reshape(2 * S // SCCH, SCCH)
    i1 = i1.reshape(2 * S // SCCH, SCCH)
    xs = _i32v_to_bf16(_dispatch(_bf16_to_i32v(x2), i0, i1))
    hs = _shared(x2, Ws1, Ws3, Ws2, hstate)
    ds = _ffn(block_expert.reshape(NBLK), xs, We_gate, We_up, We_down)
    dsv = _bf16_to_i32v(ds)
    A = _i32v_to_bf16(_gather_rows(dsv, i0))
    Bm = _i32v_to_bf16(_gather_rows(dsv, i1))
    out = _final(hs, A, Bm, w0, w1)
    return out.reshape(B, S, D)


# speedup vs baseline: 2.7142x; 2.7142x over previous
import functools

import jax
import jax.numpy as jnp
from jax.experimental import pallas as pl

B, S, D = 1, 2048, 1024
H, DH = 16, 64
E, K = 8, 2
DFF, DSH = 512, 2048
EPS = 1e-6

BS = 256
NSB = S // BS
H2 = H // 2
DP = 2 * DH
BQ = 512
NQB = S // BQ


def _rms(x, g):
    return x * jax.lax.rsqrt(jnp.mean(x * x, axis=-1, keepdims=True) + EPS) * g


def _qkv_kernel(x_ref, g_ref, wq_ref, wk_ref, wv_ref, q_ref, k_ref, v_ref):
    h = _rms(x_ref[...], g_ref[...]).astype(jnp.bfloat16)
    q = jnp.dot(h, wq_ref[...].astype(jnp.bfloat16),
                preferred_element_type=jnp.float32).astype(jnp.bfloat16)
    k = jnp.dot(h, wk_ref[...].astype(jnp.bfloat16),
                preferred_element_type=jnp.float32).astype(jnp.bfloat16)
    v = jnp.dot(h, wv_ref[...].astype(jnp.bfloat16),
                preferred_element_type=jnp.float32).astype(jnp.bfloat16)
    q_ref[...] = q.reshape(BS, H2, DP).swapaxes(0, 1)
    k_ref[...] = k.reshape(BS, H2, DP).swapaxes(0, 1)
    v_ref[...] = v.reshape(BS, H2, DP).swapaxes(0, 1)


def _qkv(x, gamma, Wq, Wk, Wv):
    g2 = gamma.reshape(1, D)
    return pl.pallas_call(
        _qkv_kernel,
        grid=(NSB,),
        in_specs=[
            pl.BlockSpec((BS, D), lambda i: (i, 0)),
            pl.BlockSpec((1, D), lambda i: (0, 0)),
            pl.BlockSpec((D, H * DH), lambda i: (0, 0)),
            pl.BlockSpec((D, H * DH), lambda i: (0, 0)),
            pl.BlockSpec((D, H * DH), lambda i: (0, 0)),
        ],
        out_specs=[
            pl.BlockSpec((H2, BS, DP), lambda i: (0, i, 0)),
            pl.BlockSpec((H2, BS, DP), lambda i: (0, i, 0)),
            pl.BlockSpec((H2, BS, DP), lambda i: (0, i, 0)),
        ],
        out_shape=[jax.ShapeDtypeStruct((H2, S, DP), jnp.bfloat16)] * 3,
    )(x, g2, Wq, Wk, Wv)


def _attn_kernel(q_ref, k_ref, v_ref, o_ref, *, scale):
    i = pl.program_id(1)
    q2 = q_ref[0]
    lane = jax.lax.broadcasted_iota(jnp.int32, (BQ, DP), 1)
    is_a = lane < DH
    zero = jnp.zeros((), jnp.bfloat16)
    qa = jnp.where(is_a, q2, zero)
    qb = jnp.where(is_a, zero, q2)
    rows = jax.lax.broadcasted_iota(jnp.int32, (BQ, BQ), 0) + i * BQ

    def step(k2, v2, carry, causal):
        ma, la, aa, mb, lb, ab = carry

        def one(qh, m, l, acc):
            s = jax.lax.dot_general(qh, k2, (((1,), (1,)), ((), ())),
                                    preferred_element_type=jnp.float32) * scale
            if causal is not None:
                s = jnp.where(causal, s, jnp.float32(-1e30))
            m_new = jnp.maximum(m, jnp.max(s, axis=-1, keepdims=True))
            alpha = jnp.exp(m - m_new)
            p = jnp.exp(s - m_new)
            l = l * alpha + jnp.sum(p, axis=-1, keepdims=True)
            acc = acc * alpha + jnp.dot(p.astype(jnp.bfloat16), v2,
                                        preferred_element_type=jnp.float32)
            return m_new, l, acc

        ma, la, aa = one(qa, ma, la, aa)
        mb, lb, ab = one(qb, mb, lb, ab)
        return ma, la, aa, mb, lb, ab

    def body(j, carry):
        base = pl.multiple_of(j * BQ, BQ)
        k2 = k_ref[0, pl.ds(base, BQ), :]
        v2 = v_ref[0, pl.ds(base, BQ), :]
        return step(k2, v2, carry, None)

    m0 = jnp.full((BQ, 1), -1e30, jnp.float32)
    l0 = jnp.zeros((BQ, 1), jnp.float32)
    a0 = jnp.zeros((BQ, DP), jnp.float32)
    carry = jax.lax.fori_loop(0, i, body, (m0, l0, a0, m0, l0, a0))
    dbase = pl.multiple_of(i * BQ, BQ)
    kd = k_ref[0, pl.ds(dbase, BQ), :]
    vd = v_ref[0, pl.ds(dbase, BQ), :]
    dcols = jax.lax.broadcasted_iota(jnp.int32, (BQ, BQ), 1)
    drows = jax.lax.broadcasted_iota(jnp.int32, (BQ, BQ), 0)
    ma, la, aa, mb, lb, ab = step(kd, vd, carry, dcols <= drows)
    oa = aa * (1.0 / la)
    ob = ab * (1.0 / lb)
    o_ref[0] = jnp.where(is_a, oa, ob).astype(jnp.bfloat16)


def _attention(q3, k3, v3):
    scale = 1.0 / float(DH) ** 0.5
    return pl.pallas_call(
        functools.partial(_attn_kernel, scale=scale),
        grid=(H2, NQB),
        in_specs=[
            pl.BlockSpec((1, BQ, DP), lambda h, i: (h, i, 0)),
            pl.BlockSpec((1, S, DP), lambda h, i: (h, 0, 0)),
            pl.BlockSpec((1, S, DP), lambda h, i: (h, 0, 0)),
        ],
        out_specs=pl.BlockSpec((1, BQ, DP), lambda h, i: (h, i, 0)),
        out_shape=jax.ShapeDtypeStruct((H2, S, DP), jnp.bfloat16),
    )(q3, k3, v3)


def _proj_router_kernel(a_ref, wo_ref, res_ref, g_ref, gw_ref,
                        h_ref, x2_ref, wfull_ref):
    attn = jnp.dot(a_ref[0], wo_ref[0].astype(jnp.bfloat16),
                   preferred_element_type=jnp.float32)
    for hh in range(1, H2):
        attn += jnp.dot(a_ref[hh], wo_ref[hh].astype(jnp.bfloat16),
                        preferred_element_type=jnp.float32)
    hstate = res_ref[...] + attn
    h_ref[...] = hstate
    x2 = _rms(hstate, g_ref[...])
    x2_ref[...] = x2.astype(jnp.bfloat16)
    logits = jnp.dot(x2, gw_ref[...], preferred_element_type=jnp.float32)
    m = jnp.max(logits, axis=-1, keepdims=True)
    p = jnp.exp(logits - m)
    p = p / jnp.sum(p, axis=-1, keepdims=True)
    idx = jax.lax.broadcasted_iota(jnp.int32, (BS, E), 1)
    m1 = jnp.max(p, axis=-1, keepdims=True)
    i1 = jnp.min(jnp.where(p == m1, idx, E), axis=-1, keepdims=True)
    p2 = jnp.where(idx == i1, -jnp.inf, p)
    m2 = jnp.max(p2, axis=-1, keepdims=True)
    i2 = jnp.min(jnp.where(p2 == m2, idx, E), axis=-1, keepdims=True)
    tot = m1 + m2
    wfull_ref[...] = jnp.where(idx == i1, m1 / tot, 0.0) + \
        jnp.where(idx == i2, m2 / tot, 0.0)


def _proj_router(attn, Wo, residual, gamma, gate_w):
    g2 = gamma.reshape(1, D)
    return pl.pallas_call(
        _proj_router_kernel,
        grid=(NSB,),
        in_specs=[
            pl.BlockSpec((H2, BS, DP), lambda i: (0, i, 0)),
            pl.BlockSpec((H2, DP, D), lambda i: (0, 0, 0)),
            pl.BlockSpec((BS, D), lambda i: (i, 0)),
            pl.BlockSpec((1, D), lambda i: (0, 0)),
            pl.BlockSpec((D, E), lambda i: (0, 0)),
        ],
        out_specs=[
            pl.BlockSpec((BS, D), lambda i: (i, 0)),
            pl.BlockSpec((BS, D), lambda i: (i, 0)),
            pl.BlockSpec((BS, E), lambda i: (i, 0)),
        ],
        out_shape=[
            jax.ShapeDtypeStruct((S, D), jnp.float32),
            jax.ShapeDtypeStruct((S, D), jnp.bfloat16),
            jax.ShapeDtypeStruct((S, E), jnp.float32),
        ],
    )(attn, Wo.reshape(H2, DP, D), residual, g2, gate_w)


def _moe_kernel(x_ref, wg_ref, wu_ref, wd_ref, w_ref, o_ref):
    e = pl.program_id(0)

    @pl.when(e == 0)
    def _():
        o_ref[...] = jnp.zeros_like(o_ref)

    x = x_ref[...]
    g = jnp.dot(x, wg_ref[0].astype(jnp.bfloat16),
                preferred_element_type=jnp.float32)
    u = jnp.dot(x, wu_ref[0].astype(jnp.bfloat16),
                preferred_element_type=jnp.float32)
    a = (g * jax.lax.logistic(g) * u).astype(jnp.bfloat16)
    d = jnp.dot(a, wd_ref[0].astype(jnp.bfloat16),
                preferred_element_type=jnp.float32)
    o_ref[...] += w_ref[0] * d


def _moe(x2, We_gate, We_up, We_down, w_full):
    wt = w_full.T.reshape(E, S, 1)
    return pl.pallas_call(
        _moe_kernel,
        grid=(E,),
        in_specs=[
            pl.BlockSpec((S, D), lambda e: (0, 0)),
            pl.BlockSpec((1, D, DFF), lambda e: (e, 0, 0)),
            pl.BlockSpec((1, D, DFF), lambda e: (e, 0, 0)),
            pl.BlockSpec((1, DFF, D), lambda e: (e, 0, 0)),
            pl.BlockSpec((1, S, 1), lambda e: (e, 0, 0)),
        ],
        out_specs=pl.BlockSpec((S, D), lambda e: (0, 0)),
        out_shape=jax.ShapeDtypeStruct((S, D), jnp.float32),
    )(x2, We_gate, We_up, We_down, wt)


def _shared_kernel(x_ref, w1_ref, w3_ref, w2_ref, h_ref, moe_ref, o_ref):
    x = x_ref[...]
    g = jnp.dot(x, w1_ref[...].astype(jnp.bfloat16),
                preferred_element_type=jnp.float32)
    u = jnp.dot(x, w3_ref[...].astype(jnp.bfloat16),
                preferred_element_type=jnp.float32)
    a = (g * jax.lax.logistic(g) * u).astype(jnp.bfloat16)
    sh = jnp.dot(a, w2_ref[...].astype(jnp.bfloat16),
                preferred_element_type=jnp.float32)
    o_ref[...] = h_ref[...] + moe_ref[...] + sh


def _shared(x2, Ws1, Ws3, Ws2, hstate, moe_out):
    return pl.pallas_call(
        _shared_kernel,
        grid=(NSB,),
        in_specs=[
            pl.BlockSpec((BS, D), lambda i: (i, 0)),
            pl.BlockSpec((D, DSH), lambda i: (0, 0)),
            pl.BlockSpec((D, DSH), lambda i: (0, 0)),
            pl.BlockSpec((DSH, D), lambda i: (0, 0)),
            pl.BlockSpec((BS, D), lambda i: (i, 0)),
            pl.BlockSpec((BS, D), lambda i: (i, 0)),
        ],
        out_specs=pl.BlockSpec((BS, D), lambda i: (i, 0)),
        out_shape=jax.ShapeDtypeStruct((S, D), jnp.float32),
    )(x2, Ws1, Ws3, Ws2, hstate, moe_out)


def kernel(hidden_states, pre_ln_gamma, post_ln_gamma, Wq, Wk, Wv, Wo,
           gate_w, We_gate, We_up, We_down, Ws1, Ws3, Ws2):
    x = hidden_states.reshape(S, D)
    q3, k3, v3 = _qkv(x, pre_ln_gamma, Wq, Wk, Wv)
    attn = _attention(q3, k3, v3)
    hstate, x2, w_full = _proj_router(attn, Wo, x, post_ln_gamma, gate_w)
    moe_out = _moe(x2, We_gate, We_up, We_down, w_full)
    out = _shared(x2, Ws1, Ws3, Ws2, hstate, moe_out)
    return out.reshape(B, S, D)


# speedup vs baseline: 2.7174x; 1.0012x over previous
import functools

import jax
import jax.numpy as jnp
from jax.experimental import pallas as pl

B, S, D = 1, 2048, 1024
H, DH = 16, 64
E, K = 8, 2
DFF, DSH = 512, 2048
EPS = 1e-6

BS = 256
NSB = S // BS
H2 = H // 2
DP = 2 * DH
BQ = 512
NQB = S // BQ


def _rms(x, g):
    return x * jax.lax.rsqrt(jnp.mean(x * x, axis=-1, keepdims=True) + EPS) * g


def _qkv_kernel(x_ref, g_ref, wq_ref, wk_ref, wv_ref, q_ref, k_ref, v_ref):
    h = _rms(x_ref[...], g_ref[...]).astype(jnp.bfloat16)
    q = jnp.dot(h, wq_ref[...].astype(jnp.bfloat16),
                preferred_element_type=jnp.float32).astype(jnp.bfloat16)
    k = jnp.dot(h, wk_ref[...].astype(jnp.bfloat16),
                preferred_element_type=jnp.float32).astype(jnp.bfloat16)
    v = jnp.dot(h, wv_ref[...].astype(jnp.bfloat16),
                preferred_element_type=jnp.float32).astype(jnp.bfloat16)
    q_ref[...] = q.reshape(BS, H2, DP).swapaxes(0, 1)
    k_ref[...] = k.reshape(BS, H2, DP).swapaxes(0, 1)
    v_ref[...] = v.reshape(BS, H2, DP).swapaxes(0, 1)


def _qkv(x, gamma, Wq, Wk, Wv):
    g2 = gamma.reshape(1, D)
    return pl.pallas_call(
        _qkv_kernel,
        grid=(NSB,),
        in_specs=[
            pl.BlockSpec((BS, D), lambda i: (i, 0)),
            pl.BlockSpec((1, D), lambda i: (0, 0)),
            pl.BlockSpec((D, H * DH), lambda i: (0, 0)),
            pl.BlockSpec((D, H * DH), lambda i: (0, 0)),
            pl.BlockSpec((D, H * DH), lambda i: (0, 0)),
        ],
        out_specs=[
            pl.BlockSpec((H2, BS, DP), lambda i: (0, i, 0)),
            pl.BlockSpec((H2, BS, DP), lambda i: (0, i, 0)),
            pl.BlockSpec((H2, BS, DP), lambda i: (0, i, 0)),
        ],
        out_shape=[jax.ShapeDtypeStruct((H2, S, DP), jnp.bfloat16)] * 3,
    )(x, g2, Wq, Wk, Wv)


def _attn_kernel(q_ref, k_ref, v_ref, o_ref, *, scale):
    i = pl.program_id(1)
    q2 = q_ref[0]
    lane = jax.lax.broadcasted_iota(jnp.int32, (BQ, DP), 1)
    is_a = lane < DH
    zero = jnp.zeros((), jnp.bfloat16)
    bscale = jnp.bfloat16(scale)
    qa = jnp.where(is_a, q2, zero) * bscale
    qb = jnp.where(is_a, zero, q2) * bscale
    rows = jax.lax.broadcasted_iota(jnp.int32, (BQ, BQ), 0) + i * BQ

    def step(k2, v2, carry, causal):
        ma, la, aa, mb, lb, ab = carry

        def one(qh, m, l, acc):
            s = jax.lax.dot_general(qh, k2, (((1,), (1,)), ((), ())),
                                    preferred_element_type=jnp.float32)
            if causal is not None:
                s = jnp.where(causal, s, jnp.float32(-1e30))
            m_new = jnp.maximum(m, jnp.max(s, axis=-1, keepdims=True))
            alpha = jnp.exp(m - m_new)
            p = jnp.exp((s - m_new).astype(jnp.bfloat16))
            l = l * alpha + jnp.sum(p, axis=-1,
                                    keepdims=True).astype(jnp.float32)
            acc = acc * alpha + jnp.dot(p, v2,
                                        preferred_element_type=jnp.float32)
            return m_new, l, acc

        ma, la, aa = one(qa, ma, la, aa)
        mb, lb, ab = one(qb, mb, lb, ab)
        return ma, la, aa, mb, lb, ab

    def body(j, carry):
        base = pl.multiple_of(j * BQ, BQ)
        k2 = k_ref[0, pl.ds(base, BQ), :]
        v2 = v_ref[0, pl.ds(base, BQ), :]
        return step(k2, v2, carry, None)

    m0 = jnp.full((BQ, 1), -1e30, jnp.float32)
    l0 = jnp.zeros((BQ, 1), jnp.float32)
    a0 = jnp.zeros((BQ, DP), jnp.float32)
    carry = jax.lax.fori_loop(0, i, body, (m0, l0, a0, m0, l0, a0))
    dbase = pl.multiple_of(i * BQ, BQ)
    kd = k_ref[0, pl.ds(dbase, BQ), :]
    vd = v_ref[0, pl.ds(dbase, BQ), :]
    dcols = jax.lax.broadcasted_iota(jnp.int32, (BQ, BQ), 1)
    drows = jax.lax.broadcasted_iota(jnp.int32, (BQ, BQ), 0)
    ma, la, aa, mb, lb, ab = step(kd, vd, carry, dcols <= drows)
    oa = aa * (1.0 / la)
    ob = ab * (1.0 / lb)
    o_ref[0] = jnp.where(is_a, oa, ob).astype(jnp.bfloat16)


def _attention(q3, k3, v3):
    scale = 1.0 / float(DH) ** 0.5
    return pl.pallas_call(
        functools.partial(_attn_kernel, scale=scale),
        grid=(H2, NQB),
        in_specs=[
            pl.BlockSpec((1, BQ, DP), lambda h, i: (h, i, 0)),
            pl.BlockSpec((1, S, DP), lambda h, i: (h, 0, 0)),
            pl.BlockSpec((1, S, DP), lambda h, i: (h, 0, 0)),
        ],
        out_specs=pl.BlockSpec((1, BQ, DP), lambda h, i: (h, i, 0)),
        out_shape=jax.ShapeDtypeStruct((H2, S, DP), jnp.bfloat16),
    )(q3, k3, v3)


def _proj_router_kernel(a_ref, wo_ref, res_ref, g_ref, gw_ref,
                        h_ref, x2_ref, wfull_ref):
    attn = jnp.dot(a_ref[0], wo_ref[0].astype(jnp.bfloat16),
                   preferred_element_type=jnp.float32)
    for hh in range(1, H2):
        attn += jnp.dot(a_ref[hh], wo_ref[hh].astype(jnp.bfloat16),
                        preferred_element_type=jnp.float32)
    hstate = res_ref[...] + attn
    h_ref[...] = hstate
    x2 = _rms(hstate, g_ref[...])
    x2_ref[...] = x2.astype(jnp.bfloat16)
    logits = jnp.dot(x2, gw_ref[...], preferred_element_type=jnp.float32)
    m = jnp.max(logits, axis=-1, keepdims=True)
    p = jnp.exp(logits - m)
    p = p / jnp.sum(p, axis=-1, keepdims=True)
    idx = jax.lax.broadcasted_iota(jnp.int32, (BS, E), 1)
    m1 = jnp.max(p, axis=-1, keepdims=True)
    i1 = jnp.min(jnp.where(p == m1, idx, E), axis=-1, keepdims=True)
    p2 = jnp.where(idx == i1, -jnp.inf, p)
    m2 = jnp.max(p2, axis=-1, keepdims=True)
    i2 = jnp.min(jnp.where(p2 == m2, idx, E), axis=-1, keepdims=True)
    tot = m1 + m2
    wfull_ref[...] = jnp.where(idx == i1, m1 / tot, 0.0) + \
        jnp.where(idx == i2, m2 / tot, 0.0)


def _proj_router(attn, Wo, residual, gamma, gate_w):
    g2 = gamma.reshape(1, D)
    return pl.pallas_call(
        _proj_router_kernel,
        grid=(NSB,),
        in_specs=[
            pl.BlockSpec((H2, BS, DP), lambda i: (0, i, 0)),
            pl.BlockSpec((H2, DP, D), lambda i: (0, 0, 0)),
            pl.BlockSpec((BS, D), lambda i: (i, 0)),
            pl.BlockSpec((1, D), lambda i: (0, 0)),
            pl.BlockSpec((D, E), lambda i: (0, 0)),
        ],
        out_specs=[
            pl.BlockSpec((BS, D), lambda i: (i, 0)),
            pl.BlockSpec((BS, D), lambda i: (i, 0)),
            pl.BlockSpec((BS, E), lambda i: (i, 0)),
        ],
        out_shape=[
            jax.ShapeDtypeStruct((S, D), jnp.float32),
            jax.ShapeDtypeStruct((S, D), jnp.bfloat16),
            jax.ShapeDtypeStruct((S, E), jnp.float32),
        ],
    )(attn, Wo.reshape(H2, DP, D), residual, g2, gate_w)


def _moe_kernel(x_ref, wg_ref, wu_ref, wd_ref, w_ref, o_ref):
    e = pl.program_id(0)

    @pl.when(e == 0)
    def _():
        o_ref[...] = jnp.zeros_like(o_ref)

    x = x_ref[...]
    g = jnp.dot(x, wg_ref[0].astype(jnp.bfloat16),
                preferred_element_type=jnp.float32)
    u = jnp.dot(x, wu_ref[0].astype(jnp.bfloat16),
                preferred_element_type=jnp.float32)
    a = (g * jax.lax.logistic(g) * u).astype(jnp.bfloat16)
    d = jnp.dot(a, wd_ref[0].astype(jnp.bfloat16),
                preferred_element_type=jnp.float32)
    o_ref[...] += w_ref[0] * d


def _moe(x2, We_gate, We_up, We_down, w_full):
    wt = w_full.T.reshape(E, S, 1)
    return pl.pallas_call(
        _moe_kernel,
        grid=(E,),
        in_specs=[
            pl.BlockSpec((S, D), lambda e: (0, 0)),
            pl.BlockSpec((1, D, DFF), lambda e: (e, 0, 0)),
            pl.BlockSpec((1, D, DFF), lambda e: (e, 0, 0)),
            pl.BlockSpec((1, DFF, D), lambda e: (e, 0, 0)),
            pl.BlockSpec((1, S, 1), lambda e: (e, 0, 0)),
        ],
        out_specs=pl.BlockSpec((S, D), lambda e: (0, 0)),
        out_shape=jax.ShapeDtypeStruct((S, D), jnp.float32),
    )(x2, We_gate, We_up, We_down, wt)


def _shared_kernel(x_ref, w1_ref, w3_ref, w2_ref, h_ref, moe_ref, o_ref):
    x = x_ref[...]
    g = jnp.dot(x, w1_ref[...].astype(jnp.bfloat16),
                preferred_element_type=jnp.float32)
    u = jnp.dot(x, w3_ref[...].astype(jnp.bfloat16),
                preferred_element_type=jnp.float32)
    a = (g * jax.lax.logistic(g) * u).astype(jnp.bfloat16)
    sh = jnp.dot(a, w2_ref[...].astype(jnp.bfloat16),
                preferred_element_type=jnp.float32)
    o_ref[...] = h_ref[...] + moe_ref[...] + sh


def _shared(x2, Ws1, Ws3, Ws2, hstate, moe_out):
    return pl.pallas_call(
        _shared_kernel,
        grid=(NSB,),
        in_specs=[
            pl.BlockSpec((BS, D), lambda i: (i, 0)),
            pl.BlockSpec((D, DSH), lambda i: (0, 0)),
            pl.BlockSpec((D, DSH), lambda i: (0, 0)),
            pl.BlockSpec((DSH, D), lambda i: (0, 0)),
            pl.BlockSpec((BS, D), lambda i: (i, 0)),
            pl.BlockSpec((BS, D), lambda i: (i, 0)),
        ],
        out_specs=pl.BlockSpec((BS, D), lambda i: (i, 0)),
        out_shape=jax.ShapeDtypeStruct((S, D), jnp.float32),
    )(x2, Ws1, Ws3, Ws2, hstate, moe_out)


def kernel(hidden_states, pre_ln_gamma, post_ln_gamma, Wq, Wk, Wv, Wo,
           gate_w, We_gate, We_up, We_down, Ws1, Ws3, Ws2):
    x = hidden_states.reshape(S, D)
    q3, k3, v3 = _qkv(x, pre_ln_gamma, Wq, Wk, Wv)
    attn = _attention(q3, k3, v3)
    hstate, x2, w_full = _proj_router(attn, Wo, x, post_ln_gamma, gate_w)
    moe_out = _moe(x2, We_gate, We_up, We_down, w_full)
    out = _shared(x2, Ws1, Ws3, Ws2, hstate, moe_out)
    return out.reshape(B, S, D)


# speedup vs baseline: 2.7411x; 1.0087x over previous
import functools

import jax
import jax.numpy as jnp
from jax.experimental import pallas as pl

B, S, D = 1, 2048, 1024
H, DH = 16, 64
E, K = 8, 2
DFF, DSH = 512, 2048
EPS = 1e-6

BS = 256
NSB = S // BS
H2 = H // 2
DP = 2 * DH
BQ = 512
NQB = S // BQ


def _rms(x, g):
    return x * jax.lax.rsqrt(jnp.mean(x * x, axis=-1, keepdims=True) + EPS) * g


def _qkv_kernel(x_ref, g_ref, wq_ref, wk_ref, wv_ref, q_ref, k_ref, v_ref):
    h = _rms(x_ref[...], g_ref[...]).astype(jnp.bfloat16)
    q = jnp.dot(h, wq_ref[...].astype(jnp.bfloat16),
                preferred_element_type=jnp.float32).astype(jnp.bfloat16)
    k = jnp.dot(h, wk_ref[...].astype(jnp.bfloat16),
                preferred_element_type=jnp.float32).astype(jnp.bfloat16)
    v = jnp.dot(h, wv_ref[...].astype(jnp.bfloat16),
                preferred_element_type=jnp.float32).astype(jnp.bfloat16)
    q_ref[...] = q.reshape(BS, H2, DP).swapaxes(0, 1)
    k_ref[...] = k.reshape(BS, H2, DP).swapaxes(0, 1)
    v_ref[...] = v.reshape(BS, H2, DP).swapaxes(0, 1)


def _qkv(x, gamma, Wq, Wk, Wv):
    g2 = gamma.reshape(1, D)
    return pl.pallas_call(
        _qkv_kernel,
        grid=(NSB,),
        in_specs=[
            pl.BlockSpec((BS, D), lambda i: (i, 0)),
            pl.BlockSpec((1, D), lambda i: (0, 0)),
            pl.BlockSpec((D, H * DH), lambda i: (0, 0)),
            pl.BlockSpec((D, H * DH), lambda i: (0, 0)),
            pl.BlockSpec((D, H * DH), lambda i: (0, 0)),
        ],
        out_specs=[
            pl.BlockSpec((H2, BS, DP), lambda i: (0, i, 0)),
            pl.BlockSpec((H2, BS, DP), lambda i: (0, i, 0)),
            pl.BlockSpec((H2, BS, DP), lambda i: (0, i, 0)),
        ],
        out_shape=[jax.ShapeDtypeStruct((H2, S, DP), jnp.bfloat16)] * 3,
    )(x, g2, Wq, Wk, Wv)


def _attn_kernel(q_ref, k_ref, v_ref, o_ref, *, scale):
    i = pl.program_id(1)
    q2 = q_ref[0]
    lane = jax.lax.broadcasted_iota(jnp.int32, (BQ, DP), 1)
    is_a = lane < DH
    zero = jnp.zeros((), jnp.bfloat16)
    bscale = jnp.bfloat16(scale)
    qa = jnp.where(is_a, q2, zero) * bscale
    qb = jnp.where(is_a, zero, q2) * bscale
    rows = jax.lax.broadcasted_iota(jnp.int32, (BQ, BQ), 0) + i * BQ

    def step(k2, v2, carry, causal):
        ma, la, aa, mb, lb, ab = carry

        def one(qh, m, l, acc):
            s = jax.lax.dot_general(qh, k2, (((1,), (1,)), ((), ())),
                                    preferred_element_type=jnp.float32)
            if causal is not None:
                s = jnp.where(causal, s, jnp.float32(-1e30))
            m_new = jnp.maximum(m, jnp.max(s, axis=-1, keepdims=True))
            alpha = jnp.exp(m - m_new)
            p = jnp.exp((s - m_new).astype(jnp.bfloat16))
            l = l * alpha + jnp.sum(p, axis=-1,
                                    keepdims=True).astype(jnp.float32)
            acc = acc * alpha + jnp.dot(p, v2,
                                        preferred_element_type=jnp.float32)
            return m_new, l, acc

        ma, la, aa = one(qa, ma, la, aa)
        mb, lb, ab = one(qb, mb, lb, ab)
        return ma, la, aa, mb, lb, ab

    def body(j, carry):
        base = pl.multiple_of(j * BQ, BQ)
        k2 = k_ref[0, pl.ds(base, BQ), :]
        v2 = v_ref[0, pl.ds(base, BQ), :]
        return step(k2, v2, carry, None)

    m0 = jnp.full((BQ, 1), -1e30, jnp.float32)
    l0 = jnp.zeros((BQ, 1), jnp.float32)
    a0 = jnp.zeros((BQ, DP), jnp.float32)
    carry = jax.lax.fori_loop(0, i, body, (m0, l0, a0, m0, l0, a0))
    dbase = pl.multiple_of(i * BQ, BQ)
    kd = k_ref[0, pl.ds(dbase, BQ), :]
    vd = v_ref[0, pl.ds(dbase, BQ), :]
    dcols = jax.lax.broadcasted_iota(jnp.int32, (BQ, BQ), 1)
    drows = jax.lax.broadcasted_iota(jnp.int32, (BQ, BQ), 0)
    ma, la, aa, mb, lb, ab = step(kd, vd, carry, dcols <= drows)
    oa = aa * (1.0 / la)
    ob = ab * (1.0 / lb)
    o_ref[0] = jnp.where(is_a, oa, ob).astype(jnp.bfloat16)


def _attention(q3, k3, v3):
    scale = 1.0 / float(DH) ** 0.5
    return pl.pallas_call(
        functools.partial(_attn_kernel, scale=scale),
        grid=(H2, NQB),
        in_specs=[
            pl.BlockSpec((1, BQ, DP), lambda h, i: (h, i, 0)),
            pl.BlockSpec((1, S, DP), lambda h, i: (h, 0, 0)),
            pl.BlockSpec((1, S, DP), lambda h, i: (h, 0, 0)),
        ],
        out_specs=pl.BlockSpec((1, BQ, DP), lambda h, i: (h, i, 0)),
        out_shape=jax.ShapeDtypeStruct((H2, S, DP), jnp.bfloat16),
    )(q3, k3, v3)


def _proj_router_kernel(a_ref, wo_ref, res_ref, g_ref, gw_ref,
                        h_ref, x2_ref, wfull_ref):
    a_flat = a_ref[...].swapaxes(0, 1).reshape(BS, H * DH)
    attn = jnp.dot(a_flat, wo_ref[...].astype(jnp.bfloat16),
                   preferred_element_type=jnp.float32)
    hstate = res_ref[...] + attn
    h_ref[...] = hstate
    x2 = _rms(hstate, g_ref[...])
    x2_ref[...] = x2.astype(jnp.bfloat16)
    logits = jnp.dot(x2, gw_ref[...], preferred_element_type=jnp.float32)
    m = jnp.max(logits, axis=-1, keepdims=True)
    p = jnp.exp(logits - m)
    p = p / jnp.sum(p, axis=-1, keepdims=True)
    idx = jax.lax.broadcasted_iota(jnp.int32, (BS, E), 1)
    m1 = jnp.max(p, axis=-1, keepdims=True)
    i1 = jnp.min(jnp.where(p == m1, idx, E), axis=-1, keepdims=True)
    p2 = jnp.where(idx == i1, -jnp.inf, p)
    m2 = jnp.max(p2, axis=-1, keepdims=True)
    i2 = jnp.min(jnp.where(p2 == m2, idx, E), axis=-1, keepdims=True)
    tot = m1 + m2
    wfull_ref[...] = jnp.where(idx == i1, m1 / tot, 0.0) + \
        jnp.where(idx == i2, m2 / tot, 0.0)


def _proj_router(attn, Wo, residual, gamma, gate_w):
    g2 = gamma.reshape(1, D)
    return pl.pallas_call(
        _proj_router_kernel,
        grid=(NSB,),
        in_specs=[
            pl.BlockSpec((H2, BS, DP), lambda i: (0, i, 0)),
            pl.BlockSpec((H * DH, D), lambda i: (0, 0)),
            pl.BlockSpec((BS, D), lambda i: (i, 0)),
            pl.BlockSpec((1, D), lambda i: (0, 0)),
            pl.BlockSpec((D, E), lambda i: (0, 0)),
        ],
        out_specs=[
            pl.BlockSpec((BS, D), lambda i: (i, 0)),
            pl.BlockSpec((BS, D), lambda i: (i, 0)),
            pl.BlockSpec((BS, E), lambda i: (i, 0)),
        ],
        out_shape=[
            jax.ShapeDtypeStruct((S, D), jnp.float32),
            jax.ShapeDtypeStruct((S, D), jnp.bfloat16),
            jax.ShapeDtypeStruct((S, E), jnp.float32),
        ],
    )(attn, Wo, residual, g2, gate_w)


def _moe_kernel(x_ref, wg_ref, wu_ref, wd_ref, w_ref, o_ref):
    e = pl.program_id(0)

    @pl.when(e == 0)
    def _():
        o_ref[...] = jnp.zeros_like(o_ref)

    x = x_ref[...]
    g = jnp.dot(x, wg_ref[0].astype(jnp.bfloat16),
                preferred_element_type=jnp.float32)
    u = jnp.dot(x, wu_ref[0].astype(jnp.bfloat16),
                preferred_element_type=jnp.float32)
    a = (g * jax.lax.logistic(g) * u).astype(jnp.bfloat16)
    d = jnp.dot(a, wd_ref[0].astype(jnp.bfloat16),
                preferred_element_type=jnp.float32)
    o_ref[...] += w_ref[0] * d


def _moe(x2, We_gate, We_up, We_down, w_full):
    wt = w_full.T.reshape(E, S, 1)
    return pl.pallas_call(
        _moe_kernel,
        grid=(E,),
        in_specs=[
            pl.BlockSpec((S, D), lambda e: (0, 0)),
            pl.BlockSpec((1, D, DFF), lambda e: (e, 0, 0)),
            pl.BlockSpec((1, D, DFF), lambda e: (e, 0, 0)),
            pl.BlockSpec((1, DFF, D), lambda e: (e, 0, 0)),
            pl.BlockSpec((1, S, 1), lambda e: (e, 0, 0)),
        ],
        out_specs=pl.BlockSpec((S, D), lambda e: (0, 0)),
        out_shape=jax.ShapeDtypeStruct((S, D), jnp.float32),
    )(x2, We_gate, We_up, We_down, wt)


def _shared_kernel(x_ref, w1_ref, w3_ref, w2_ref, h_ref, moe_ref, o_ref):
    x = x_ref[...]
    g = jnp.dot(x, w1_ref[...].astype(jnp.bfloat16),
                preferred_element_type=jnp.float32)
    u = jnp.dot(x, w3_ref[...].astype(jnp.bfloat16),
                preferred_element_type=jnp.float32)
    a = (g * jax.lax.logistic(g) * u).astype(jnp.bfloat16)
    sh = jnp.dot(a, w2_ref[...].astype(jnp.bfloat16),
                preferred_element_type=jnp.float32)
    o_ref[...] = h_ref[...] + moe_ref[...] + sh


def _shared(x2, Ws1, Ws3, Ws2, hstate, moe_out):
    return pl.pallas_call(
        _shared_kernel,
        grid=(NSB,),
        in_specs=[
            pl.BlockSpec((BS, D), lambda i: (i, 0)),
            pl.BlockSpec((D, DSH), lambda i: (0, 0)),
            pl.BlockSpec((D, DSH), lambda i: (0, 0)),
            pl.BlockSpec((DSH, D), lambda i: (0, 0)),
            pl.BlockSpec((BS, D), lambda i: (i, 0)),
            pl.BlockSpec((BS, D), lambda i: (i, 0)),
        ],
        out_specs=pl.BlockSpec((BS, D), lambda i: (i, 0)),
        out_shape=jax.ShapeDtypeStruct((S, D), jnp.float32),
    )(x2, Ws1, Ws3, Ws2, hstate, moe_out)


def kernel(hidden_states, pre_ln_gamma, post_ln_gamma, Wq, Wk, Wv, Wo,
           gate_w, We_gate, We_up, We_down, Ws1, Ws3, Ws2):
    x = hidden_states.reshape(S, D)
    q3, k3, v3 = _qkv(x, pre_ln_gamma, Wq, Wk, Wv)
    attn = _attention(q3, k3, v3)
    hstate, x2, w_full = _proj_router(attn, Wo, x, post_ln_gamma, gate_w)
    moe_out = _moe(x2, We_gate, We_up, We_down, w_full)
    out = _shared(x2, Ws1, Ws3, Ws2, hstate, moe_out)
    return out.reshape(B, S, D)
